# Initial kernel scaffold; baseline (speedup 1.0000x reference)
import jax
import jax.numpy as jnp
from jax.experimental import pallas as pl


def kernel(x_1, edge_index_1, x_2, edge_index_2, W0, b0, ln0_g, ln0_b, Wg0, bg0, lng0, lnb0, Wg1, bg1, lng1, lnb1):
    raise NotImplementedError("write your pallas kernel here")


# speedup vs baseline: 18.6384x; 18.6384x over previous
import functools

import jax
import jax.numpy as jnp
from jax import lax
from jax.experimental import pallas as pl
from jax.experimental.pallas import tpu as pltpu
from jax.experimental.pallas import tpu_sc as plsc

N = 10000
E = 320000
DIN = 256
F = 128

R = 10240
RT = 2 * R
NT = 16
CH = 157
EPT = CH * 128
EPB = NT * EPT
RPT = R // NT
CB = 32


def _degree_body(dst_hbm, ones_hbm, deg_out, dst_v, ones_v, acc):
    c = lax.axis_index("c")
    s = lax.axis_index("s")
    w = c * NT + s
    pltpu.sync_copy(ones_hbm.at[pl.ds(0, 128)], ones_v)
    pltpu.sync_copy(dst_hbm.at[w], dst_v)
    pltpu.sync_copy(ones_hbm, acc.at[pl.ds(s * RPT, RPT)])
    plsc.subcore_barrier()

    def body(j, carry):
        pltpu.sync_copy(ones_v, acc.at[dst_v.at[j]], add=True)
        return carry

    lax.fori_loop(0, CH, body, 0)
    plsc.subcore_barrier()
    pltpu.sync_copy(acc.at[pl.ds(s * RPT, RPT)],
                    deg_out.at[pl.ds(c * R + s * RPT, RPT)])


def _conv_body(hp_hbm, src_hbm, dst_hbm, s_out, src_v, dst_v, buf, acc, sem):
    c = lax.axis_index("c")
    s = lax.axis_index("s")
    w = c * NT + s
    pltpu.sync_copy(hp_hbm.at[pl.ds(c * R + s * RPT, RPT)],
                    acc.at[pl.ds(s * RPT, RPT)])
    plsc.subcore_barrier()
    for b in range(0, CH, CB):
        nj = min(CB, CH - b)
        pltpu.sync_copy(src_hbm.at[w, pl.ds(b, nj)], src_v.at[pl.ds(0, nj)])
        pltpu.sync_copy(dst_hbm.at[w, pl.ds(b, nj)], dst_v.at[pl.ds(0, nj)])

        def body(j, carry):
            pltpu.async_copy(hp_hbm.at[src_v.at[j]], buf, sem).wait()
            pltpu.sync_copy(buf, acc.at[dst_v.at[j]], add=True)
            return carry

        lax.fori_loop(0, nj, body, 0)
    plsc.subcore_barrier()
    pltpu.sync_copy(acc.at[pl.ds(s * RPT, RPT)],
                    s_out.at[pl.ds(c * R + s * RPT, RPT)])


@functools.cache
def _sc_kernels():
    mesh = plsc.VectorSubcoreMesh(core_axis_name="c", subcore_axis_name="s")
    deg_k = functools.partial(
        pl.kernel,
        mesh=mesh,
        out_type=jax.ShapeDtypeStruct((RT,), jnp.float32),
        scratch_types=[
            pltpu.VMEM((CH, 128), jnp.int32),
            pltpu.VMEM((128,), jnp.float32),
            pltpu.VMEM_SHARED((R,), jnp.float32),
        ],
    )(_degree_body)
    conv_k = functools.partial(
        pl.kernel,
        mesh=mesh,
        out_type=jax.ShapeDtypeStruct((RT, F), jnp.float32),
        scratch_types=[
            pltpu.VMEM((CB, 128), jnp.int32),
            pltpu.VMEM((CB, 128), jnp.int32),
            pltpu.VMEM((128, F), jnp.float32),
            pltpu.VMEM_SHARED((R, F), jnp.float32),
            pltpu.SemaphoreType.DMA,
        ],
    )(_conv_body)
    return deg_k, conv_k


def _sc_degree(dst_g, ones):
    return _sc_kernels()[0](dst_g, ones)


def _sc_conv(hp, src_g, dst_g):
    return _sc_kernels()[1](hp, src_g, dst_g)


_BR = 1024


def _ln_leaky(h, g, b):
    m = jnp.mean(h, axis=1, keepdims=True)
    d = h - m
    v = jnp.mean(d * d, axis=1, keepdims=True)
    hn = d * lax.rsqrt(v + 1e-5) * g + b
    return jnp.where(hn >= 0, hn, 0.01 * hn)


def _stage0_body(x_ref, w0t_ref, b0_ref, g_ref, bb_ref, wgt_ref, deg_ref,
                 out_ref):
    h0 = jnp.dot(x_ref[...], w0t_ref[...],
                 preferred_element_type=jnp.float32) + b0_ref[...]
    hl = _ln_leaky(h0, g_ref[...], bb_ref[...])
    h = jnp.dot(hl, wgt_ref[...], preferred_element_type=jnp.float32)
    dinv = lax.rsqrt(jnp.maximum(deg_ref[...], 1.0))
    out_ref[...] = h * dinv


def _stage1_body(s_ref, deg_ref, bg_ref, g_ref, bb_ref, wgt_ref, out_ref):
    dinv = lax.rsqrt(jnp.maximum(deg_ref[...], 1.0))
    h0 = s_ref[...] * dinv + bg_ref[...]
    hl = _ln_leaky(h0, g_ref[...], bb_ref[...])
    h = jnp.dot(hl, wgt_ref[...], preferred_element_type=jnp.float32)
    out_ref[...] = h * dinv


def _stage2_body(s_ref, deg_ref, bg_ref, g_ref, bb_ref, out_ref):
    dinv = lax.rsqrt(jnp.maximum(deg_ref[...], 1.0))
    h0 = s_ref[...] * dinv + bg_ref[...]
    out_ref[...] = _ln_leaky(h0, g_ref[...], bb_ref[...])


def _row_spec(width):
    return pl.BlockSpec((_BR, width), lambda i: (i, 0))


def _full_spec(shape):
    return pl.BlockSpec(shape, lambda i: (0,) * len(shape))


def _tc_stage0(x, w0t, b0, g, bb, wgt, deg):
    return pl.pallas_call(
        _stage0_body,
        grid=(RT // _BR,),
        in_specs=[
            _row_spec(DIN),
            _full_spec((DIN, F)),
            _full_spec((1, F)),
            _full_spec((1, F)),
            _full_spec((1, F)),
            _full_spec((F, F)),
            _row_spec(1),
        ],
        out_specs=_row_spec(F),
        out_shape=jax.ShapeDtypeStruct((RT, F), jnp.float32),
    )(x, w0t, b0, g, bb, wgt, deg)


def _tc_stage1(s, deg, bg, g, bb, wgt):
    return pl.pallas_call(
        _stage1_body,
        grid=(RT // _BR,),
        in_specs=[
            _row_spec(F),
            _row_spec(1),
            _full_spec((1, F)),
            _full_spec((1, F)),
            _full_spec((1, F)),
            _full_spec((F, F)),
        ],
        out_specs=_row_spec(F),
        out_shape=jax.ShapeDtypeStruct((RT, F), jnp.float32),
    )(s, deg, bg, g, bb, wgt)


def _tc_stage2(s, deg, bg, g, bb):
    return pl.pallas_call(
        _stage2_body,
        grid=(RT // _BR,),
        in_specs=[
            _row_spec(F),
            _row_spec(1),
            _full_spec((1, F)),
            _full_spec((1, F)),
            _full_spec((1, F)),
        ],
        out_specs=_row_spec(F),
        out_shape=jax.ShapeDtypeStruct((RT, F), jnp.float32),
    )(s, deg, bg, g, bb)


def _prep_edges(edge_index, branch):
    ofs = branch * R
    pad = EPB - E
    idx = jnp.arange(pad, dtype=jnp.int32)
    src = jnp.concatenate([edge_index[0] + ofs, (idx % N) + ofs])
    dst = jnp.concatenate([edge_index[1], N + (idx % (R - N))])
    return src.reshape(NT, CH, 128), dst.reshape(NT, CH, 128)


def kernel(x_1, edge_index_1, x_2, edge_index_2, W0, b0, ln0_g, ln0_b,
           Wg0, bg0, lng0, lnb0, Wg1, bg1, lng1, lnb1):
    x = jnp.zeros((RT, DIN), jnp.float32)
    x = x.at[0:N].set(x_1).at[R:R + N].set(x_2)
    src1, dst1 = _prep_edges(edge_index_1, 0)
    src2, dst2 = _prep_edges(edge_index_2, 1)
    src_g = jnp.concatenate([src1, src2]).reshape(2 * NT, CH, 128)
    dst_g = jnp.concatenate([dst1, dst2]).reshape(2 * NT, CH, 128)
    ones = jnp.ones((RPT,), jnp.float32)

    w0t = W0.T
    wg0t = Wg0.T
    wg1t = Wg1.T
    row = lambda v: v.reshape(1, F)

    deg = _sc_degree(dst_g, ones).reshape(RT, 1)
    hp0 = _tc_stage0(x, w0t, row(b0), row(ln0_g), row(ln0_b), wg0t, deg)
    s0 = _sc_conv(hp0, src_g, dst_g)
    hp1 = _tc_stage1(s0, deg, row(bg0), row(lng0), row(lnb0), wg1t)
    s1 = _sc_conv(hp1, src_g, dst_g)
    y = _tc_stage2(s1, deg, row(bg1), row(lng1), row(lnb1))

    return (y[0:N], y[R:R + N])


# speedup vs baseline: 27.2153x; 1.4602x over previous
import functools

import jax
import jax.numpy as jnp
from jax import lax
from jax.experimental import pallas as pl
from jax.experimental.pallas import tpu as pltpu
from jax.experimental.pallas import tpu_sc as plsc

N = 10000
E = 320000
DIN = 256
F = 128

R = 10240
RT = 2 * R
NT = 16
CH = 157
EPT = CH * 128
EPB = NT * EPT
RPT = R // NT
CB = 32


def _degree_body(dst_hbm, ones_hbm, deg_out, dst_v, ones_v, acc):
    c = lax.axis_index("c")
    s = lax.axis_index("s")
    w = c * NT + s
    pltpu.sync_copy(ones_hbm.at[pl.ds(0, 128)], ones_v)
    pltpu.sync_copy(dst_hbm.at[w], dst_v)
    pltpu.sync_copy(ones_hbm, acc.at[pl.ds(s * RPT, RPT)])
    plsc.subcore_barrier()

    def body(j, carry):
        pltpu.sync_copy(ones_v, acc.at[dst_v.at[j]], add=True)
        return carry

    lax.fori_loop(0, CH, body, 0)
    plsc.subcore_barrier()
    pltpu.sync_copy(acc.at[pl.ds(s * RPT, RPT)],
                    deg_out.at[pl.ds(c * R + s * RPT, RPT)])


def _conv_body(hp_hbm, src_hbm, dst_hbm, s_out, src_v, dst_v, buf, buf2, acc,
               sem, sem2):
    c = lax.axis_index("c")
    s = lax.axis_index("s")
    w = c * NT + s
    pltpu.sync_copy(hp_hbm.at[pl.ds(c * R + s * RPT, RPT)],
                    acc.at[pl.ds(s * RPT, RPT)])
    plsc.subcore_barrier()
    for b in range(0, CH, CB):
        nj = min(CB, CH - b)
        pltpu.sync_copy(src_hbm.at[w, pl.ds(b, nj)], src_v.at[pl.ds(0, nj)])
        pltpu.sync_copy(dst_hbm.at[w, pl.ds(b, nj)], dst_v.at[pl.ds(0, nj)])
        pltpu.async_copy(hp_hbm.at[src_v.at[0]], buf, sem)

        def pair(g, carry):
            j0 = 2 * g

            @pl.when(j0 + 1 < nj)
            def _():
                pltpu.async_copy(hp_hbm.at[src_v.at[j0 + 1]], buf2, sem2)

            pltpu.make_async_copy(hp_hbm.at[src_v.at[j0]], buf, sem).wait()
            pltpu.sync_copy(buf, acc.at[dst_v.at[j0]], add=True)

            @pl.when(j0 + 2 < nj)
            def _():
                pltpu.async_copy(hp_hbm.at[src_v.at[j0 + 2]], buf, sem)

            @pl.when(j0 + 1 < nj)
            def _():
                pltpu.make_async_copy(hp_hbm.at[src_v.at[j0 + 1]], buf2,
                                      sem2).wait()
                pltpu.sync_copy(buf2, acc.at[dst_v.at[j0 + 1]], add=True)

            return carry

        lax.fori_loop(0, (nj + 1) // 2, pair, 0)
    plsc.subcore_barrier()
    pltpu.sync_copy(acc.at[pl.ds(s * RPT, RPT)],
                    s_out.at[pl.ds(c * R + s * RPT, RPT)])


@functools.cache
def _sc_kernels():
    mesh = plsc.VectorSubcoreMesh(core_axis_name="c", subcore_axis_name="s")
    deg_k = functools.partial(
        pl.kernel,
        mesh=mesh,
        out_type=jax.ShapeDtypeStruct((RT,), jnp.float32),
        scratch_types=[
            pltpu.VMEM((CH, 128), jnp.int32),
            pltpu.VMEM((128,), jnp.float32),
            pltpu.VMEM_SHARED((R,), jnp.float32),
        ],
    )(_degree_body)
    conv_k = functools.partial(
        pl.kernel,
        mesh=mesh,
        out_type=jax.ShapeDtypeStruct((RT, F), jnp.float32),
        scratch_types=[
            pltpu.VMEM((CB, 128), jnp.int32),
            pltpu.VMEM((CB, 128), jnp.int32),
            pltpu.VMEM((128, F), jnp.float32),
            pltpu.VMEM((128, F), jnp.float32),
            pltpu.VMEM_SHARED((R, F), jnp.float32),
            pltpu.SemaphoreType.DMA,
            pltpu.SemaphoreType.DMA,
        ],
    )(_conv_body)
    return deg_k, conv_k


def _sc_degree(dst_g, ones):
    return _sc_kernels()[0](dst_g, ones)


def _sc_conv(hp, src_g, dst_g):
    return _sc_kernels()[1](hp, src_g, dst_g)


_BR = 1024


def _ln_leaky(h, g, b):
    m = jnp.mean(h, axis=1, keepdims=True)
    d = h - m
    v = jnp.mean(d * d, axis=1, keepdims=True)
    hn = d * lax.rsqrt(v + 1e-5) * g + b
    return jnp.where(hn >= 0, hn, 0.01 * hn)


def _stage0_body(x_ref, w0t_ref, b0_ref, g_ref, bb_ref, wgt_ref, deg_ref,
                 out_ref):
    h0 = jnp.dot(x_ref[...], w0t_ref[...],
                 preferred_element_type=jnp.float32) + b0_ref[...]
    hl = _ln_leaky(h0, g_ref[...], bb_ref[...])
    h = jnp.dot(hl, wgt_ref[...], preferred_element_type=jnp.float32)
    dinv = lax.rsqrt(jnp.maximum(deg_ref[...], 1.0))
    out_ref[...] = h * dinv


def _stage1_body(s_ref, deg_ref, bg_ref, g_ref, bb_ref, wgt_ref, out_ref):
    dinv = lax.rsqrt(jnp.maximum(deg_ref[...], 1.0))
    h0 = s_ref[...] * dinv + bg_ref[...]
    hl = _ln_leaky(h0, g_ref[...], bb_ref[...])
    h = jnp.dot(hl, wgt_ref[...], preferred_element_type=jnp.float32)
    out_ref[...] = h * dinv


def _stage2_body(s_ref, deg_ref, bg_ref, g_ref, bb_ref, out_ref):
    dinv = lax.rsqrt(jnp.maximum(deg_ref[...], 1.0))
    h0 = s_ref[...] * dinv + bg_ref[...]
    out_ref[...] = _ln_leaky(h0, g_ref[...], bb_ref[...])


def _row_spec(width):
    return pl.BlockSpec((_BR, width), lambda i: (i, 0))


def _full_spec(shape):
    return pl.BlockSpec(shape, lambda i: (0,) * len(shape))


def _tc_stage0(x, w0t, b0, g, bb, wgt, deg):
    return pl.pallas_call(
        _stage0_body,
        grid=(RT // _BR,),
        in_specs=[
            _row_spec(DIN),
            _full_spec((DIN, F)),
            _full_spec((1, F)),
            _full_spec((1, F)),
            _full_spec((1, F)),
            _full_spec((F, F)),
            _row_spec(1),
        ],
        out_specs=_row_spec(F),
        out_shape=jax.ShapeDtypeStruct((RT, F), jnp.float32),
    )(x, w0t, b0, g, bb, wgt, deg)


def _tc_stage1(s, deg, bg, g, bb, wgt):
    return pl.pallas_call(
        _stage1_body,
        grid=(RT // _BR,),
        in_specs=[
            _row_spec(F),
            _row_spec(1),
            _full_spec((1, F)),
            _full_spec((1, F)),
            _full_spec((1, F)),
            _full_spec((F, F)),
        ],
        out_specs=_row_spec(F),
        out_shape=jax.ShapeDtypeStruct((RT, F), jnp.float32),
    )(s, deg, bg, g, bb, wgt)


def _tc_stage2(s, deg, bg, g, bb):
    return pl.pallas_call(
        _stage2_body,
        grid=(RT // _BR,),
        in_specs=[
            _row_spec(F),
            _row_spec(1),
            _full_spec((1, F)),
            _full_spec((1, F)),
            _full_spec((1, F)),
        ],
        out_specs=_row_spec(F),
        out_shape=jax.ShapeDtypeStruct((RT, F), jnp.float32),
    )(s, deg, bg, g, bb)


def _prep_edges(edge_index, branch):
    ofs = branch * R
    pad = EPB - E
    idx = jnp.arange(pad, dtype=jnp.int32)
    src = jnp.concatenate([edge_index[0] + ofs, (idx % N) + ofs])
    dst = jnp.concatenate([edge_index[1], N + (idx % (R - N))])
    return src.reshape(NT, CH, 128), dst.reshape(NT, CH, 128)


def kernel(x_1, edge_index_1, x_2, edge_index_2, W0, b0, ln0_g, ln0_b,
           Wg0, bg0, lng0, lnb0, Wg1, bg1, lng1, lnb1):
    x = jnp.zeros((RT, DIN), jnp.float32)
    x = x.at[0:N].set(x_1).at[R:R + N].set(x_2)
    src1, dst1 = _prep_edges(edge_index_1, 0)
    src2, dst2 = _prep_edges(edge_index_2, 1)
    src_g = jnp.concatenate([src1, src2]).reshape(2 * NT, CH, 128)
    dst_g = jnp.concatenate([dst1, dst2]).reshape(2 * NT, CH, 128)
    ones = jnp.ones((RPT,), jnp.float32)

    w0t = W0.T
    wg0t = Wg0.T
    wg1t = Wg1.T
    row = lambda v: v.reshape(1, F)

    deg = _sc_degree(dst_g, ones).reshape(RT, 1)
    hp0 = _tc_stage0(x, w0t, row(b0), row(ln0_g), row(ln0_b), wg0t, deg)
    s0 = _sc_conv(hp0, src_g, dst_g)
    hp1 = _tc_stage1(s0, deg, row(bg0), row(lng0), row(lnb0), wg1t)
    s1 = _sc_conv(hp1, src_g, dst_g)
    y = _tc_stage2(s1, deg, row(bg1), row(lng1), row(lnb1))

    return (y[0:N], y[R:R + N])
